# Initial kernel scaffold; baseline (speedup 1.0000x reference)
import functools

import jax
import jax.numpy as jnp
from jax import lax
from jax.experimental import pallas as pl
from jax.experimental.pallas import tpu as pltpu
from jax.experimental.pallas import tpu_sc as plsc

N = 10000
LANE = 128
NB = 79
NCHUNK = 32
EPAD = NCHUNK * NB * LANE
NTRASH = 64
NACC = N + NTRASH
ROWS_A, ROWS_LAST = 632, N - 15 * 632
DEG_LAST = NACC - 15 * ROWS_A

_mesh = plsc.VectorSubcoreMesh(core_axis_name="c", subcore_axis_name="s")


def _fill(ref, n16, val):
    for i in range(n16):
        ref[pl.ds(16 * i, 16)] = jnp.zeros((16,), jnp.float32) + val


@functools.partial(
    pl.kernel,
    mesh=_mesh,
    out_type=jax.ShapeDtypeStruct((2, N), jnp.float32),
    scratch_types=[
        pltpu.VMEM((NB, LANE), jnp.int32),
        pltpu.VMEM((640,), jnp.float32),
        pltpu.VMEM((640,), jnp.float32),
        pltpu.VMEM_SHARED((NACC,), jnp.float32),
    ],
)
def _deg(dst_hbm, deg_out, idx_d, ones_v, init_v, deg_sh):
    cid = lax.axis_index("c")
    tid = lax.axis_index("s")
    _fill(ones_v, 40, 1.0)
    _fill(init_v, 40, jnp.where(cid == 0, 1.0, 0.0).astype(jnp.float32))
    pltpu.sync_copy(dst_hbm.at[cid * 16 + tid], idx_d)

    @pl.when(tid < 15)
    def _():
        pltpu.sync_copy(init_v.at[pl.ds(0, ROWS_A)],
                        deg_sh.at[pl.ds(tid * ROWS_A, ROWS_A)])

    @pl.when(tid == 15)
    def _():
        pltpu.sync_copy(init_v.at[pl.ds(0, DEG_LAST)],
                        deg_sh.at[pl.ds(15 * ROWS_A, DEG_LAST)])

    plsc.subcore_barrier()

    def body(b, carry):
        pltpu.sync_copy(ones_v.at[pl.ds(0, LANE)],
                        deg_sh.at[idx_d.at[b]], add=True)
        return carry

    lax.fori_loop(0, NB, body, 0)
    plsc.subcore_barrier()

    @pl.when(tid < 15)
    def _():
        pltpu.sync_copy(deg_sh.at[pl.ds(tid * ROWS_A, ROWS_A)],
                        deg_out.at[cid, pl.ds(tid * ROWS_A, ROWS_A)])

    @pl.when(tid == 15)
    def _():
        pltpu.sync_copy(deg_sh.at[pl.ds(15 * ROWS_A, ROWS_LAST)],
                        deg_out.at[cid, pl.ds(15 * ROWS_A, ROWS_LAST)])


def _make_prop_fsplit(dh):
    @functools.partial(
        pl.kernel,
        mesh=_mesh,
        out_type=jax.ShapeDtypeStruct((N, 2 * dh), jnp.float32),
        scratch_types=[
            pltpu.VMEM((2, NB, LANE), jnp.int32),
            pltpu.VMEM((2, NB, LANE), jnp.int32),
            pltpu.VMEM((LANE, dh), jnp.float32),
            pltpu.VMEM_SHARED((NACC, dh), jnp.float32),
            pltpu.SemaphoreType.DMA,
        ],
    )
    def prop(tbl0, tbl1, src_hbm, dst_hbm, out, idx_s, idx_d, rowbuf, acc, gsem):
        cid = lax.axis_index("c")
        tid = lax.axis_index("s")
        pltpu.sync_copy(src_hbm.at[tid], idx_s.at[0])
        pltpu.sync_copy(src_hbm.at[tid + 16], idx_s.at[1])
        pltpu.sync_copy(dst_hbm.at[tid], idx_d.at[0])
        pltpu.sync_copy(dst_hbm.at[tid + 16], idx_d.at[1])

        def run(tbl, col):
            @pl.when(tid < 15)
            def _():
                pltpu.sync_copy(tbl.at[pl.ds(tid * ROWS_A, ROWS_A)],
                                acc.at[pl.ds(tid * ROWS_A, ROWS_A)])

            @pl.when(tid == 15)
            def _():
                pltpu.sync_copy(tbl.at[pl.ds(15 * ROWS_A, ROWS_LAST)],
                                acc.at[pl.ds(15 * ROWS_A, ROWS_LAST)])

            plsc.subcore_barrier()
            for j in range(2):
                def body(b, carry):
                    pltpu.async_copy(tbl.at[idx_s.at[j, b]], rowbuf, gsem).wait()
                    pltpu.sync_copy(rowbuf, acc.at[idx_d.at[j, b]], add=True)
                    return carry

                lax.fori_loop(0, NB, body, 0)
            plsc.subcore_barrier()

            @pl.when(tid < 15)
            def _():
                pltpu.sync_copy(acc.at[pl.ds(tid * ROWS_A, ROWS_A)],
                                out.at[pl.ds(tid * ROWS_A, ROWS_A), pl.ds(col, dh)])

            @pl.when(tid == 15)
            def _():
                pltpu.sync_copy(acc.at[pl.ds(15 * ROWS_A, ROWS_LAST)],
                                out.at[pl.ds(15 * ROWS_A, ROWS_LAST), pl.ds(col, dh)])

        @pl.when(cid == 0)
        def _():
            run(tbl0, 0)

        @pl.when(cid == 1)
        def _():
            run(tbl1, dh)

    return prop


_prop64 = _make_prop_fsplit(64)
_prop128 = _make_prop_fsplit(128)


@functools.partial(
    pl.kernel,
    mesh=_mesh,
    out_type=jax.ShapeDtypeStruct((2, N, 16), jnp.float32),
    scratch_types=[
        pltpu.VMEM((NB, LANE), jnp.int32),
        pltpu.VMEM((NB, LANE), jnp.int32),
        pltpu.VMEM((LANE, 16), jnp.float32),
        pltpu.VMEM_SHARED((NACC, 16), jnp.float32),
        pltpu.SemaphoreType.DMA,
    ],
)
def _prop16(tbl, src_hbm, dst_hbm, out, idx_s, idx_d, rowbuf, acc, gsem):
    cid = lax.axis_index("c")
    tid = lax.axis_index("s")
    w = cid * 16 + tid
    pltpu.sync_copy(src_hbm.at[w], idx_s)
    pltpu.sync_copy(dst_hbm.at[w], idx_d)

    @pl.when(tid < 15)
    def _():
        pltpu.sync_copy(tbl.at[pl.ds(tid * ROWS_A, ROWS_A)],
                        acc.at[pl.ds(tid * ROWS_A, ROWS_A)])

    @pl.when(tid == 15)
    def _():
        pltpu.sync_copy(tbl.at[pl.ds(15 * ROWS_A, ROWS_LAST)],
                        acc.at[pl.ds(15 * ROWS_A, ROWS_LAST)])

    plsc.subcore_barrier()

    def body(b, carry):
        pltpu.async_copy(tbl.at[idx_s.at[b]], rowbuf, gsem).wait()
        pltpu.sync_copy(rowbuf, acc.at[idx_d.at[b]], add=True)
        return carry

    lax.fori_loop(0, NB, body, 0)
    plsc.subcore_barrier()

    @pl.when(tid < 15)
    def _():
        pltpu.sync_copy(acc.at[pl.ds(tid * ROWS_A, ROWS_A)],
                        out.at[cid, pl.ds(tid * ROWS_A, ROWS_A), :])

    @pl.when(tid == 15)
    def _():
        pltpu.sync_copy(acc.at[pl.ds(15 * ROWS_A, ROWS_LAST)],
                        out.at[cid, pl.ds(15 * ROWS_A, ROWS_LAST), :])


R = 2000
GRID = N // R


def _row_spec(w):
    return pl.BlockSpec((R, w), lambda i: (i, 0))


def _full_spec(shape):
    nd = len(shape)
    return pl.BlockSpec(shape, lambda i: (0,) * nd)


def _s1_body(deg_ref, x_ref, dinv_ref, xs0_ref, xs1_ref):
    d = deg_ref[:, 0:1] + deg_ref[:, 1:2]
    dinv = lax.rsqrt(d)
    dinv_ref[...] = dinv
    xs = x_ref[...] * dinv
    xs0_ref[...] = xs[:, :64]
    xs1_ref[...] = xs[:, 64:]


def _s1(degT, x):
    return pl.pallas_call(
        _s1_body,
        grid=(GRID,),
        in_specs=[_row_spec(2), _row_spec(128)],
        out_specs=[_row_spec(1), _row_spec(64), _row_spec(64)],
        out_shape=[
            jax.ShapeDtypeStruct((N, 1), jnp.float32),
            jax.ShapeDtypeStruct((N, 64), jnp.float32),
            jax.ShapeDtypeStruct((N, 64), jnp.float32),
        ],
    )(degT, x)


def _t1_body(a_ref, dinv_ref, w_ref, b_ref, z_ref, st_ref):
    zb = jnp.dot(a_ref[...] * dinv_ref[...], w_ref[...],
                 preferred_element_type=jnp.float32) + b_ref[...]
    z_ref[...] = zb
    st = jnp.concatenate(
        [jnp.sum(zb, 0, keepdims=True), jnp.sum(zb * zb, 0, keepdims=True)], 0)

    @pl.when(pl.program_id(0) == 0)
    def _():
        st_ref[...] = st

    @pl.when(pl.program_id(0) > 0)
    def _():
        st_ref[...] += st


def _t1(a, dinv, w, b):
    din, dout = w.shape
    return pl.pallas_call(
        _t1_body,
        grid=(GRID,),
        in_specs=[_row_spec(din), _row_spec(1),
                  _full_spec((din, dout)), _full_spec((1, dout))],
        out_specs=[_row_spec(dout), _full_spec((2, dout))],
        out_shape=[
            jax.ShapeDtypeStruct((N, dout), jnp.float32),
            jax.ShapeDtypeStruct((2, dout), jnp.float32),
        ],
    )(a, dinv, w, b)


def _bn_relu(z, st_ref, g_ref, be_ref):
    mu = st_ref[0:1, :] * (1.0 / N)
    var = st_ref[1:2, :] * (1.0 / N) - mu * mu
    inv = lax.rsqrt(var + 1e-5)
    return jnp.maximum((z - mu) * (inv * g_ref[...]) + be_ref[...], 0.0)


def _t2a_body(z_ref, st_ref, g_ref, be_ref, dinv_ref, h_ref, hs0_ref, hs1_ref):
    hb = _bn_relu(z_ref[...], st_ref, g_ref, be_ref)
    h_ref[...] = hb
    hs = hb * dinv_ref[...]
    hs0_ref[...] = hs[:, :128]
    hs1_ref[...] = hs[:, 128:]


def _t2a(z, st, g, be, dinv):
    return pl.pallas_call(
        _t2a_body,
        grid=(GRID,),
        in_specs=[_row_spec(256), _full_spec((2, 256)),
                  _full_spec((1, 256)), _full_spec((1, 256)), _row_spec(1)],
        out_specs=[_row_spec(256), _row_spec(128), _row_spec(128)],
        out_shape=[
            jax.ShapeDtypeStruct((N, 256), jnp.float32),
            jax.ShapeDtypeStruct((N, 128), jnp.float32),
            jax.ShapeDtypeStruct((N, 128), jnp.float32),
        ],
    )(z, st, g, be, dinv)


def _t2b_body(z_ref, st_ref, g_ref, be_ref, dinv_ref, h1_ref, w3_ref, y3_ref):
    h2 = _bn_relu(z_ref[...], st_ref, g_ref, be_ref) + h1_ref[...]
    y3_ref[...] = jnp.dot(h2 * dinv_ref[...], w3_ref[...],
                          preferred_element_type=jnp.float32)


def _t2b(z, st, g, be, dinv, h1, w3p):
    return pl.pallas_call(
        _t2b_body,
        grid=(GRID,),
        in_specs=[_row_spec(256), _full_spec((2, 256)),
                  _full_spec((1, 256)), _full_spec((1, 256)), _row_spec(1),
                  _row_spec(256), _full_spec((256, 16))],
        out_specs=_row_spec(16),
        out_shape=jax.ShapeDtypeStruct((N, 16), jnp.float32),
    )(z, st, g, be, dinv, h1, w3p)


def _s7_body(p_ref, y3_ref, dinv_ref, b3_ref, o_ref):
    o_ref[...] = ((p_ref[0] + p_ref[1] - y3_ref[...]) * dinv_ref[...]
                  + b3_ref[...])


def _s7(p, y3, dinv, b3p):
    return pl.pallas_call(
        _s7_body,
        grid=(GRID,),
        in_specs=[pl.BlockSpec((2, R, 16), lambda i: (0, i, 0)),
                  _row_spec(16), _row_spec(1), _full_spec((1, 16))],
        out_specs=_row_spec(16),
        out_shape=jax.ShapeDtypeStruct((N, 16), jnp.float32),
    )(p, y3, dinv, b3p)


def kernel(x, edge_index, W1, b1, g1, be1, W2, b2, g2, be2, W3, b3):
    e = edge_index.shape[1]
    pad = EPAD - e
    src = edge_index[0].astype(jnp.int32)
    dst = edge_index[1].astype(jnp.int32)
    pad_src = (jnp.arange(pad, dtype=jnp.int32) * LANE) % N
    pad_dst = N + jnp.arange(pad, dtype=jnp.int32) % NTRASH
    src_r = jnp.concatenate([src, pad_src]).reshape(NCHUNK, NB, LANE)
    dst_r = jnp.concatenate([dst, pad_dst]).reshape(NCHUNK, NB, LANE)

    deg2 = _deg(dst_r)
    dinv, xs0, xs1 = _s1(deg2.T, x)
    a1 = _prop64(xs0, xs1, src_r, dst_r)
    z1, st1 = _t1(a1, dinv, W1, b1.reshape(1, -1))
    h, hs0, hs1 = _t2a(z1, st1, g1.reshape(1, -1), be1.reshape(1, -1), dinv)
    a2 = _prop128(hs0, hs1, src_r, dst_r)
    z2, st2 = _t1(a2, dinv, W2, b2.reshape(1, -1))
    w3p = jnp.pad(W3, ((0, 0), (0, 14)))
    y3 = _t2b(z2, st2, g2.reshape(1, -1), be2.reshape(1, -1), dinv, h, w3p)
    p = _prop16(y3, src_r, dst_r)
    b3p = jnp.pad(b3, (0, 14)).reshape(1, 16)
    o16 = _s7(p, y3, dinv, b3p)
    return o16[:, :2]


# speedup vs baseline: 16.9680x; 16.9680x over previous
import functools

import jax
import jax.numpy as jnp
from jax import lax
from jax.experimental import pallas as pl
from jax.experimental.pallas import tpu as pltpu
from jax.experimental.pallas import tpu_sc as plsc

N = 10000
ROWS = 640
N_UP = 16 * ROWS
LANE = 128
NB = 79
NCHUNK = 32
EPAD = NCHUNK * NB * LANE
DH = 128

_mesh = plsc.VectorSubcoreMesh(core_axis_name="c", subcore_axis_name="s")


def _fill(ref, n16, val):
    for i in range(n16):
        ref[pl.ds(16 * i, 16)] = jnp.zeros((16,), jnp.float32) + val


@functools.partial(
    pl.kernel,
    mesh=_mesh,
    out_type=[jax.ShapeDtypeStruct((N_UP,), jnp.float32),
              jax.ShapeDtypeStruct((N_UP,), jnp.float32)],
    scratch_types=[
        pltpu.VMEM((NB, LANE), jnp.int32),
        pltpu.VMEM((ROWS,), jnp.float32),
        pltpu.VMEM((ROWS,), jnp.float32),
        pltpu.VMEM_SHARED((N_UP,), jnp.float32),
    ],
)
def _deg(dst_hbm, deg0_out, deg1_out, idx_d, ones_v, init_v, deg_sh):
    cid = lax.axis_index("c")
    tid = lax.axis_index("s")
    _fill(ones_v, ROWS // 16, 1.0)
    _fill(init_v, ROWS // 16, jnp.where(cid == 0, 1.0, 0.0).astype(jnp.float32))
    pltpu.sync_copy(dst_hbm.at[cid * 16 + tid], idx_d)
    pltpu.sync_copy(init_v, deg_sh.at[pl.ds(tid * ROWS, ROWS)])
    plsc.subcore_barrier()

    def body(b, carry):
        pltpu.sync_copy(ones_v.at[pl.ds(0, LANE)],
                        deg_sh.at[idx_d.at[b]], add=True)
        return carry

    lax.fori_loop(0, NB, body, 0)
    plsc.subcore_barrier()

    @pl.when(cid == 0)
    def _():
        pltpu.sync_copy(deg_sh.at[pl.ds(tid * ROWS, ROWS)],
                        deg0_out.at[pl.ds(tid * ROWS, ROWS)])

    @pl.when(cid == 1)
    def _():
        pltpu.sync_copy(deg_sh.at[pl.ds(tid * ROWS, ROWS)],
                        deg1_out.at[pl.ds(tid * ROWS, ROWS)])


def _make_prop(edge_split):
    ncnk = 1 if edge_split else 2

    @functools.partial(
        pl.kernel,
        mesh=_mesh,
        out_type=[jax.ShapeDtypeStruct((N_UP, DH), jnp.float32),
                  jax.ShapeDtypeStruct((N_UP, DH), jnp.float32)],
        scratch_types=[
            pltpu.VMEM((ncnk, NB, LANE), jnp.int32),
            pltpu.VMEM((ncnk, NB, LANE), jnp.int32),
            pltpu.VMEM((LANE, DH), jnp.float32),
            pltpu.VMEM_SHARED((N_UP, DH), jnp.float32),
            pltpu.SemaphoreType.DMA,
        ],
    )
    def prop(tbl0, tbl1, src_hbm, dst_hbm, out0, out1, idx_s, idx_d, rowbuf,
             acc, gsem):
        cid = lax.axis_index("c")
        tid = lax.axis_index("s")
        if edge_split:
            pltpu.sync_copy(src_hbm.at[cid * 16 + tid], idx_s.at[0])
            pltpu.sync_copy(dst_hbm.at[cid * 16 + tid], idx_d.at[0])
        else:
            pltpu.sync_copy(src_hbm.at[tid], idx_s.at[0])
            pltpu.sync_copy(src_hbm.at[tid + 16], idx_s.at[1])
            pltpu.sync_copy(dst_hbm.at[tid], idx_d.at[0])
            pltpu.sync_copy(dst_hbm.at[tid + 16], idx_d.at[1])

        def run(tbl, out):
            pltpu.sync_copy(tbl.at[pl.ds(tid * ROWS, ROWS)],
                            acc.at[pl.ds(tid * ROWS, ROWS)])
            plsc.subcore_barrier()
            for j in range(ncnk):
                def body(b, carry):
                    pltpu.async_copy(tbl.at[idx_s.at[j, b]], rowbuf, gsem).wait()
                    pltpu.sync_copy(rowbuf, acc.at[idx_d.at[j, b]], add=True)
                    return carry

                lax.fori_loop(0, NB, body, 0)
            plsc.subcore_barrier()
            pltpu.sync_copy(acc.at[pl.ds(tid * ROWS, ROWS)],
                            out.at[pl.ds(tid * ROWS, ROWS), :])

        @pl.when(cid == 0)
        def _():
            run(tbl0, out0)

        @pl.when(cid == 1)
        def _():
            run(tbl1, out1)

    return prop


_prop_esplit = _make_prop(True)


R = 2048
GRID = N_UP // R


def _row_spec(w):
    return pl.BlockSpec((R, w), lambda i: (i, 0))


def _full_spec(shape):
    nd = len(shape)
    return pl.BlockSpec(shape, lambda i: (0,) * nd)


def _s1_body(deg0_ref, deg1_ref, x_ref, dinv_ref, xs_ref):
    d = deg0_ref[...] + deg1_ref[...]
    dinv = lax.rsqrt(d)
    dinv_ref[...] = dinv
    xs_ref[...] = x_ref[...] * dinv


def _s1(deg0, deg1, x):
    return pl.pallas_call(
        _s1_body,
        grid=(GRID,),
        in_specs=[_row_spec(1), _row_spec(1), _row_spec(128)],
        out_specs=[_row_spec(1), _row_spec(128)],
        out_shape=[
            jax.ShapeDtypeStruct((N_UP, 1), jnp.float32),
            jax.ShapeDtypeStruct((N_UP, 128), jnp.float32),
        ],
    )(deg0, deg1, x)


def _stats_update(zb, st_ref):
    row = pl.program_id(0) * R + lax.broadcasted_iota(jnp.int32, zb.shape, 0)
    zm = jnp.where(row < N, zb, 0.0)
    st = jnp.concatenate(
        [jnp.sum(zm, 0, keepdims=True), jnp.sum(zm * zm, 0, keepdims=True)], 0)

    @pl.when(pl.program_id(0) == 0)
    def _():
        st_ref[...] = st

    @pl.when(pl.program_id(0) > 0)
    def _():
        st_ref[...] += st


def _t1a_body(a0_ref, a1_ref, xs_ref, dinv_ref, w_ref, b_ref, z_ref, st_ref):
    a = a0_ref[...] + a1_ref[...] - xs_ref[...]
    zb = jnp.dot(a * dinv_ref[...], w_ref[...],
                 preferred_element_type=jnp.float32) + b_ref[...]
    z_ref[...] = zb
    _stats_update(zb, st_ref)


def _t1a(a0, a1, xs, dinv, w, b):
    din, dout = w.shape
    return pl.pallas_call(
        _t1a_body,
        grid=(GRID,),
        in_specs=[_row_spec(din), _row_spec(din), _row_spec(din),
                  _row_spec(1), _full_spec((din, dout)), _full_spec((1, dout))],
        out_specs=[_row_spec(dout), _full_spec((2, dout))],
        out_shape=[
            jax.ShapeDtypeStruct((N_UP, dout), jnp.float32),
            jax.ShapeDtypeStruct((2, dout), jnp.float32),
        ],
    )(a0, a1, xs, dinv, w, b)


def _t1b_body(a00_ref, a01_ref, hs0_ref, a10_ref, a11_ref, hs1_ref,
              dinv_ref, w_ref, b_ref, z_ref, st_ref):
    a = jnp.concatenate(
        [a00_ref[...] + a01_ref[...] - hs0_ref[...],
         a10_ref[...] + a11_ref[...] - hs1_ref[...]], axis=1)
    zb = jnp.dot(a * dinv_ref[...], w_ref[...],
                 preferred_element_type=jnp.float32) + b_ref[...]
    z_ref[...] = zb
    _stats_update(zb, st_ref)


def _t1b(parts, dinv, w, b):
    din, dout = w.shape
    return pl.pallas_call(
        _t1b_body,
        grid=(GRID,),
        in_specs=[_row_spec(din // 2)] * 6 + [_row_spec(1),
                  _full_spec((din, dout)), _full_spec((1, dout))],
        out_specs=[_row_spec(dout), _full_spec((2, dout))],
        out_shape=[
            jax.ShapeDtypeStruct((N_UP, dout), jnp.float32),
            jax.ShapeDtypeStruct((2, dout), jnp.float32),
        ],
    )(*parts, dinv, w, b)


def _bn_relu(z, st_ref, g_ref, be_ref):
    mu = st_ref[0:1, :] * (1.0 / N)
    var = st_ref[1:2, :] * (1.0 / N) - mu * mu
    inv = lax.rsqrt(var + 1e-5)
    return jnp.maximum((z - mu) * (inv * g_ref[...]) + be_ref[...], 0.0)


def _t2a_body(z_ref, st_ref, g_ref, be_ref, dinv_ref, h_ref, hs0_ref, hs1_ref):
    hb = _bn_relu(z_ref[...], st_ref, g_ref, be_ref)
    h_ref[...] = hb
    hs = hb * dinv_ref[...]
    hs0_ref[...] = hs[:, :128]
    hs1_ref[...] = hs[:, 128:]


def _t2a(z, st, g, be, dinv):
    return pl.pallas_call(
        _t2a_body,
        grid=(GRID,),
        in_specs=[_row_spec(256), _full_spec((2, 256)),
                  _full_spec((1, 256)), _full_spec((1, 256)), _row_spec(1)],
        out_specs=[_row_spec(256), _row_spec(128), _row_spec(128)],
        out_shape=[
            jax.ShapeDtypeStruct((N_UP, 256), jnp.float32),
            jax.ShapeDtypeStruct((N_UP, 128), jnp.float32),
            jax.ShapeDtypeStruct((N_UP, 128), jnp.float32),
        ],
    )(z, st, g, be, dinv)


def _t2b_body(z_ref, st_ref, g_ref, be_ref, dinv_ref, h1_ref, w3_ref, y3_ref):
    h2 = _bn_relu(z_ref[...], st_ref, g_ref, be_ref) + h1_ref[...]
    y3_ref[...] = jnp.dot(h2 * dinv_ref[...], w3_ref[...],
                          preferred_element_type=jnp.float32)


def _t2b(z, st, g, be, dinv, h1, w3p):
    return pl.pallas_call(
        _t2b_body,
        grid=(GRID,),
        in_specs=[_row_spec(256), _full_spec((2, 256)),
                  _full_spec((1, 256)), _full_spec((1, 256)), _row_spec(1),
                  _row_spec(256), _full_spec((256, DH))],
        out_specs=_row_spec(DH),
        out_shape=jax.ShapeDtypeStruct((N_UP, DH), jnp.float32),
    )(z, st, g, be, dinv, h1, w3p)


def _s7_body(p0_ref, p1_ref, y3_ref, dinv_ref, b3_ref, o_ref):
    o_ref[...] = ((p0_ref[...] + p1_ref[...] - y3_ref[...]) * dinv_ref[...]
                  + b3_ref[...])


def _s7(p0, p1, y3, dinv, b3p):
    return pl.pallas_call(
        _s7_body,
        grid=(GRID,),
        in_specs=[_row_spec(DH), _row_spec(DH),
                  _row_spec(DH), _row_spec(1), _full_spec((1, DH))],
        out_specs=_row_spec(DH),
        out_shape=jax.ShapeDtypeStruct((N_UP, DH), jnp.float32),
    )(p0, p1, y3, dinv, b3p)


def kernel(x, edge_index, W1, b1, g1, be1, W2, b2, g2, be2, W3, b3):
    e = edge_index.shape[1]
    pad = EPAD - e
    src = edge_index[0].astype(jnp.int32)
    dst = edge_index[1].astype(jnp.int32)
    pad_src = (jnp.arange(pad, dtype=jnp.int32) * LANE) % N
    pad_dst = N + jnp.arange(pad, dtype=jnp.int32) % (N_UP - N)
    src_r = jnp.concatenate([src, pad_src]).reshape(NCHUNK, NB, LANE)
    dst_r = jnp.concatenate([dst, pad_dst]).reshape(NCHUNK, NB, LANE)
    xp = jnp.pad(x, ((0, N_UP - N), (0, 0)))

    deg0, deg1 = _deg(dst_r)
    dinv, xs = _s1(deg0.reshape(N_UP, 1), deg1.reshape(N_UP, 1), xp)
    a10, a11 = _prop_esplit(xs, xs, src_r, dst_r)
    z1, st1 = _t1a(a10, a11, xs, dinv, W1, b1.reshape(1, -1))
    h, hs0, hs1 = _t2a(z1, st1, g1.reshape(1, -1), be1.reshape(1, -1), dinv)
    a200, a201 = _prop_esplit(hs0, hs0, src_r, dst_r)
    a210, a211 = _prop_esplit(hs1, hs1, src_r, dst_r)
    z2, st2 = _t1b((a200, a201, hs0, a210, a211, hs1),
                   dinv, W2, b2.reshape(1, -1))
    w3p = jnp.pad(W3, ((0, 0), (0, DH - W3.shape[1])))
    y3 = _t2b(z2, st2, g2.reshape(1, -1), be2.reshape(1, -1), dinv, h, w3p)
    p0, p1 = _prop_esplit(y3, y3, src_r, dst_r)
    b3p = jnp.pad(b3, (0, DH - b3.shape[0])).reshape(1, DH)
    o16 = _s7(p0, p1, y3, dinv, b3p)
    return o16[:N, :2]
